# Initial kernel scaffold; baseline (speedup 1.0000x reference)
import jax
import jax.numpy as jnp
from jax.experimental import pallas as pl


def kernel(positions, coords, features):
    raise NotImplementedError("write your pallas kernel here")


# speedup vs baseline: 1.7763x; 1.7763x over previous
import functools

import jax
import jax.numpy as jnp
from jax import lax
from jax.experimental import pallas as pl
from jax.experimental.pallas import tpu as pltpu

_K = 8
_BANDWIDTH = 0.05
_TEMP = 2.0 * _BANDWIDTH ** 2 + 1e-08
_BIG_ID = 2 ** 30


def _topk_body(pt_ref, q_ref, v_ref, i_ref, *, n_real, cblk, n_chunks):
    q = q_ref[...]
    qblk = q.shape[0]
    qn = (q[:, 0:1] * q[:, 0:1] + q[:, 1:2] * q[:, 1:2]
          + q[:, 2:3] * q[:, 2:3])
    q8 = jnp.concatenate([q, jnp.zeros((qblk, 5), jnp.float32)],
                         axis=1).astype(jnp.bfloat16)

    def step(c, carry):
        bv, bi = carry
        base = c * cblk
        pch = pt_ref[:, pl.ds(base, cblk)]
        pn = pch[0] * pch[0] + pch[1] * pch[1] + pch[2] * pch[2]
        p8 = jnp.concatenate([pch, jnp.zeros((5, cblk), jnp.float32)],
                             axis=0).astype(jnp.bfloat16)
        cross = lax.dot_general(q8, p8, (((1,), (0,)), ((), ())),
                                preferred_element_type=jnp.float32)
        d = jnp.maximum(qn + pn[None, :] - 2.0 * cross, 0.0)
        ids = base + lax.broadcasted_iota(jnp.int32, (qblk, cblk), 1)
        d = jnp.where(ids < n_real, d, jnp.inf)
        allv = jnp.concatenate([bv, d], axis=1)
        alli = jnp.concatenate([bi, ids], axis=1)
        vs, idxs = [], []
        for _ in range(_K):
            m = jnp.min(allv, axis=1, keepdims=True)
            am = jnp.min(jnp.where(allv == m, alli, jnp.full_like(alli, _BIG_ID)),
                         axis=1, keepdims=True)
            vs.append(m)
            idxs.append(am)
            allv = jnp.where(alli == am, jnp.inf, allv)
        return jnp.concatenate(vs, axis=1), jnp.concatenate(idxs, axis=1)

    bv0 = jnp.full((qblk, _K), jnp.inf, dtype=jnp.float32)
    bi0 = _BIG_ID + lax.broadcasted_iota(jnp.int32, (qblk, _K), 1)
    bv, bi = lax.fori_loop(0, n_chunks, step, (bv0, bi0))
    v_ref[...] = bv
    i_ref[...] = bi


def _run_topk(positions, coords):
    n = positions.shape[0]
    qn = coords.shape[0]
    cblk = 2048 if n >= 2048 else 128
    n_chunks = -(-n // cblk)
    npad = n_chunks * cblk
    pt = jnp.pad(jnp.transpose(positions), ((0, 0), (0, npad - n)))
    qblk = 128 if qn % 128 == 0 else qn
    grid = (qn // qblk,)
    body = functools.partial(_topk_body, n_real=n, cblk=cblk,
                             n_chunks=n_chunks)
    return pl.pallas_call(
        body,
        grid=grid,
        in_specs=[
            pl.BlockSpec((3, npad), lambda g: (0, 0)),
            pl.BlockSpec((qblk, 3), lambda g: (g, 0)),
        ],
        out_specs=[
            pl.BlockSpec((qblk, _K), lambda g: (g, 0)),
            pl.BlockSpec((qblk, _K), lambda g: (g, 0)),
        ],
        out_shape=[
            jax.ShapeDtypeStruct((qn, _K), jnp.float32),
            jax.ShapeDtypeStruct((qn, _K), jnp.int32),
        ],
        compiler_params=pltpu.CompilerParams(
            dimension_semantics=("arbitrary",)),
    )(pt, coords)


def kernel(positions, coords, features):
    positions = positions.astype(jnp.float32)
    coords = coords.astype(jnp.float32)
    topv, topi = _run_topk(positions, coords)
    w = jax.nn.softmax(-topv / _TEMP, axis=1)
    out = jnp.sum(w[..., None] * features[topi], axis=1)
    return out.astype(coords.dtype)


# speedup vs baseline: 1.8828x; 1.0599x over previous
import functools

import jax
import jax.numpy as jnp
from jax import lax
from jax.experimental import pallas as pl
from jax.experimental.pallas import tpu as pltpu

_K = 8
_BANDWIDTH = 0.05
_TEMP = 2.0 * _BANDWIDTH ** 2 + 1e-08
_INIT_ID = 1.0e7
_BIG_ID = 2.0e7


def _extract8(allv, alli):
    vs, idxs = [], []
    for _ in range(_K):
        m = jnp.min(allv, axis=1, keepdims=True)
        am = jnp.min(jnp.where(allv == m, alli, jnp.full_like(alli, _BIG_ID)),
                     axis=1, keepdims=True)
        vs.append(m)
        idxs.append(am)
        allv = jnp.where(alli == am, jnp.inf, allv)
    return jnp.concatenate(vs, axis=1), jnp.concatenate(idxs, axis=1)


def _topk_body(pt_ref, pb_ref, q_ref, v_ref, i_ref, *, n_real, cblk,
               n_chunks):
    q = q_ref[...]
    qblk = q.shape[0]
    qn = (q[:, 0:1] * q[:, 0:1] + q[:, 1:2] * q[:, 1:2]
          + q[:, 2:3] * q[:, 2:3])
    q8 = jnp.concatenate([q, jnp.zeros((qblk, 5), jnp.float32)],
                         axis=1).astype(jnp.bfloat16)

    def step(c, carry):
        bv, bi = carry
        base = c * cblk
        pch = pt_ref[:, pl.ds(base, cblk)]
        pn = pch[0] * pch[0] + pch[1] * pch[1] + pch[2] * pch[2]
        cross = lax.dot_general(q8, pb_ref[:, pl.ds(base, cblk)],
                                (((1,), (0,)), ((), ())),
                                preferred_element_type=jnp.float32)
        d = jnp.maximum(qn + pn[None, :] - 2.0 * cross, 0.0)
        ids = base + lax.broadcasted_iota(jnp.int32, (qblk, cblk),
                                          1).astype(jnp.float32)
        d = jnp.where(ids < n_real, d, jnp.inf)
        cv, ci = _extract8(d, ids)
        mv, mi = _extract8(jnp.concatenate([bv, cv], axis=1),
                           jnp.concatenate([bi, ci], axis=1))
        return mv, mi

    bv0 = jnp.full((qblk, _K), jnp.inf, dtype=jnp.float32)
    bi0 = _INIT_ID + lax.broadcasted_iota(jnp.int32, (qblk, _K),
                                          1).astype(jnp.float32)
    bv, bi = lax.fori_loop(0, n_chunks, step, (bv0, bi0))
    v_ref[...] = bv
    i_ref[...] = bi.astype(jnp.int32)


def _run_topk(positions, coords):
    n = positions.shape[0]
    qn = coords.shape[0]
    cblk = 2048 if n >= 2048 else 128
    n_chunks = -(-n // cblk)
    npad = n_chunks * cblk
    pt = jnp.pad(jnp.transpose(positions), ((0, 0), (0, npad - n)))
    pb = jnp.pad(pt, ((0, 5), (0, 0))).astype(jnp.bfloat16)
    qblk = 128 if qn % 128 == 0 else qn
    grid = (qn // qblk,)
    body = functools.partial(_topk_body, n_real=n, cblk=cblk,
                             n_chunks=n_chunks)
    return pl.pallas_call(
        body,
        grid=grid,
        in_specs=[
            pl.BlockSpec((3, npad), lambda g: (0, 0)),
            pl.BlockSpec((8, npad), lambda g: (0, 0)),
            pl.BlockSpec((qblk, 3), lambda g: (g, 0)),
        ],
        out_specs=[
            pl.BlockSpec((qblk, _K), lambda g: (g, 0)),
            pl.BlockSpec((qblk, _K), lambda g: (g, 0)),
        ],
        out_shape=[
            jax.ShapeDtypeStruct((qn, _K), jnp.float32),
            jax.ShapeDtypeStruct((qn, _K), jnp.int32),
        ],
        compiler_params=pltpu.CompilerParams(
            dimension_semantics=("arbitrary",)),
    )(pt, pb, coords)


def kernel(positions, coords, features):
    positions = positions.astype(jnp.float32)
    coords = coords.astype(jnp.float32)
    topv, topi = _run_topk(positions, coords)
    w = jax.nn.softmax(-topv / _TEMP, axis=1)
    out = jnp.sum(w[..., None] * features[topi], axis=1)
    return out.astype(coords.dtype)


# speedup vs baseline: 2.3564x; 1.2516x over previous
import functools

import jax
import jax.numpy as jnp
from jax import lax
from jax.experimental import pallas as pl
from jax.experimental.pallas import tpu as pltpu

_K = 8
_BANDWIDTH = 0.05
_TEMP = 2.0 * _BANDWIDTH ** 2 + 1e-08
_INIT_ID = 1.0e7
_BIG_ID = 2.0e7


def _extract8(allv, alli):
    vs, idxs = [], []
    for _ in range(_K):
        m = jnp.min(allv, axis=1, keepdims=True)
        am = jnp.min(jnp.where(allv == m, alli, jnp.full_like(alli, _BIG_ID)),
                     axis=1, keepdims=True)
        vs.append(m)
        idxs.append(am)
        allv = jnp.where(alli == am, jnp.inf, allv)
    return jnp.concatenate(vs, axis=1), jnp.concatenate(idxs, axis=1)


def _topk_body(pt_ref, pb_ref, q_ref, v_ref, i_ref, *, n_real, cblk,
               n_chunks):
    q = q_ref[...]
    qblk = q.shape[0]
    qn = (q[:, 0:1] * q[:, 0:1] + q[:, 1:2] * q[:, 1:2]
          + q[:, 2:3] * q[:, 2:3])
    q8 = jnp.concatenate([q, jnp.zeros((qblk, 5), jnp.float32)],
                         axis=1).astype(jnp.bfloat16)

    def step(c, carry):
        bv, bi = carry
        base = c * cblk
        pch = pt_ref[:, pl.ds(base, cblk)]
        pn = pch[0] * pch[0] + pch[1] * pch[1] + pch[2] * pch[2]
        cross = lax.dot_general(q8, pb_ref[:, pl.ds(base, cblk)],
                                (((1,), (0,)), ((), ())),
                                preferred_element_type=jnp.float32)
        d = jnp.maximum(qn + pn[None, :] - 2.0 * cross, 0.0)
        ids = base + lax.broadcasted_iota(jnp.int32, (qblk, cblk),
                                          1).astype(jnp.float32)
        d = jnp.where(ids < n_real, d, jnp.inf)
        cv, ci = _extract8(d, ids)
        mv, mi = _extract8(jnp.concatenate([bv, cv], axis=1),
                           jnp.concatenate([bi, ci], axis=1))
        return mv, mi

    bv0 = jnp.full((qblk, _K), jnp.inf, dtype=jnp.float32)
    bi0 = _INIT_ID + lax.broadcasted_iota(jnp.int32, (qblk, _K),
                                          1).astype(jnp.float32)
    bv, bi = lax.fori_loop(0, n_chunks, step, (bv0, bi0))
    v_ref[...] = bv
    i_ref[...] = bi.astype(jnp.int32)


def _run_topk(positions, coords):
    n = positions.shape[0]
    qn = coords.shape[0]
    cblk = 4096 if n >= 4096 else 128
    n_chunks = -(-n // cblk)
    npad = n_chunks * cblk
    pt = jnp.pad(jnp.transpose(positions), ((0, 0), (0, npad - n)))
    pb = jnp.pad(pt, ((0, 5), (0, 0))).astype(jnp.bfloat16)
    qblk = 128 if qn % 128 == 0 else qn
    grid = (qn // qblk,)
    body = functools.partial(_topk_body, n_real=n, cblk=cblk,
                             n_chunks=n_chunks)
    return pl.pallas_call(
        body,
        grid=grid,
        in_specs=[
            pl.BlockSpec((3, npad), lambda g: (0, 0)),
            pl.BlockSpec((8, npad), lambda g: (0, 0)),
            pl.BlockSpec((qblk, 3), lambda g: (g, 0)),
        ],
        out_specs=[
            pl.BlockSpec((qblk, _K), lambda g: (g, 0)),
            pl.BlockSpec((qblk, _K), lambda g: (g, 0)),
        ],
        out_shape=[
            jax.ShapeDtypeStruct((qn, _K), jnp.float32),
            jax.ShapeDtypeStruct((qn, _K), jnp.int32),
        ],
        compiler_params=pltpu.CompilerParams(
            dimension_semantics=("arbitrary",)),
    )(pt, pb, coords)


def kernel(positions, coords, features):
    positions = positions.astype(jnp.float32)
    coords = coords.astype(jnp.float32)
    topv, topi = _run_topk(positions, coords)
    w = jax.nn.softmax(-topv / _TEMP, axis=1)
    out = jnp.sum(w[..., None] * features[topi], axis=1)
    return out.astype(coords.dtype)


# speedup vs baseline: 2.4922x; 1.0576x over previous
import functools

import jax
import jax.numpy as jnp
from jax import lax
from jax.experimental import pallas as pl
from jax.experimental.pallas import tpu as pltpu

_K = 8
_BANDWIDTH = 0.05
_TEMP = 2.0 * _BANDWIDTH ** 2 + 1e-08
_INIT_ID = 1.0e7
_BIG_ID = 2.0e7


def _extract8(allv, alli):
    vs, idxs = [], []
    for _ in range(_K):
        m = jnp.min(allv, axis=1, keepdims=True)
        am = jnp.min(jnp.where(allv == m, alli, jnp.full_like(alli, _BIG_ID)),
                     axis=1, keepdims=True)
        vs.append(m)
        idxs.append(am)
        allv = jnp.where(alli == am, jnp.inf, allv)
    return jnp.concatenate(vs, axis=1), jnp.concatenate(idxs, axis=1)


def _topk_body(pt_ref, pb_ref, q_ref, v_ref, i_ref, *, n_real, cblk,
               n_chunks):
    q = q_ref[...]
    qblk = q.shape[0]
    qn = (q[:, 0:1] * q[:, 0:1] + q[:, 1:2] * q[:, 1:2]
          + q[:, 2:3] * q[:, 2:3])
    q8 = jnp.concatenate([q, jnp.zeros((qblk, 5), jnp.float32)],
                         axis=1).astype(jnp.bfloat16)

    def step(c, carry):
        bv, bi = carry
        base = c * cblk
        pch = pt_ref[:, pl.ds(base, cblk)]
        pn = pch[0] * pch[0] + pch[1] * pch[1] + pch[2] * pch[2]
        cross = lax.dot_general(q8, pb_ref[:, pl.ds(base, cblk)],
                                (((1,), (0,)), ((), ())),
                                preferred_element_type=jnp.float32)
        d = jnp.maximum(qn + pn[None, :] - 2.0 * cross, 0.0)
        ids = base + lax.broadcasted_iota(jnp.int32, (qblk, cblk),
                                          1).astype(jnp.float32)
        d = jnp.where(ids < n_real, d, jnp.inf)
        cv, ci = _extract8(d, ids)
        mv, mi = _extract8(jnp.concatenate([bv, cv], axis=1),
                           jnp.concatenate([bi, ci], axis=1))
        return mv, mi

    bv0 = jnp.full((qblk, _K), jnp.inf, dtype=jnp.float32)
    bi0 = _INIT_ID + lax.broadcasted_iota(jnp.int32, (qblk, _K),
                                          1).astype(jnp.float32)
    bv, bi = lax.fori_loop(0, n_chunks, step, (bv0, bi0))
    v_ref[...] = bv
    i_ref[...] = bi.astype(jnp.int32)


def _run_topk(positions, coords):
    n = positions.shape[0]
    qn = coords.shape[0]
    cblk = 8192 if n >= 8192 else 128
    n_chunks = -(-n // cblk)
    npad = n_chunks * cblk
    pt = jnp.pad(jnp.transpose(positions), ((0, 0), (0, npad - n)))
    pb = jnp.pad(pt, ((0, 5), (0, 0))).astype(jnp.bfloat16)
    qblk = 128 if qn % 128 == 0 else qn
    grid = (qn // qblk,)
    body = functools.partial(_topk_body, n_real=n, cblk=cblk,
                             n_chunks=n_chunks)
    return pl.pallas_call(
        body,
        grid=grid,
        in_specs=[
            pl.BlockSpec((3, npad), lambda g: (0, 0)),
            pl.BlockSpec((8, npad), lambda g: (0, 0)),
            pl.BlockSpec((qblk, 3), lambda g: (g, 0)),
        ],
        out_specs=[
            pl.BlockSpec((qblk, _K), lambda g: (g, 0)),
            pl.BlockSpec((qblk, _K), lambda g: (g, 0)),
        ],
        out_shape=[
            jax.ShapeDtypeStruct((qn, _K), jnp.float32),
            jax.ShapeDtypeStruct((qn, _K), jnp.int32),
        ],
        compiler_params=pltpu.CompilerParams(
            dimension_semantics=("arbitrary",)),
    )(pt, pb, coords)


def kernel(positions, coords, features):
    positions = positions.astype(jnp.float32)
    coords = coords.astype(jnp.float32)
    topv, topi = _run_topk(positions, coords)
    w = jax.nn.softmax(-topv / _TEMP, axis=1)
    out = jnp.sum(w[..., None] * features[topi], axis=1)
    return out.astype(coords.dtype)


# speedup vs baseline: 2.6234x; 1.0526x over previous
import functools

import jax
import jax.numpy as jnp
from jax import lax
from jax.experimental import pallas as pl
from jax.experimental.pallas import tpu as pltpu

_K = 8
_BANDWIDTH = 0.05
_TEMP = 2.0 * _BANDWIDTH ** 2 + 1e-08
_INIT_ID = 1.0e7
_BIG_ID = 2.0e7


def _extract8(allv, alli):
    vs, idxs = [], []
    for _ in range(_K):
        m = jnp.min(allv, axis=1, keepdims=True)
        am = jnp.min(jnp.where(allv == m, alli, jnp.full_like(alli, _BIG_ID)),
                     axis=1, keepdims=True)
        vs.append(m)
        idxs.append(am)
        allv = jnp.where(alli == am, jnp.inf, allv)
    return jnp.concatenate(vs, axis=1), jnp.concatenate(idxs, axis=1)


def _topk_body(pt_ref, pb_ref, q_ref, v_ref, i_ref, *, n_real, cblk,
               n_chunks):
    q = q_ref[...]
    qblk = q.shape[0]
    qn = (q[:, 0:1] * q[:, 0:1] + q[:, 1:2] * q[:, 1:2]
          + q[:, 2:3] * q[:, 2:3])
    q8 = jnp.concatenate([q, jnp.zeros((qblk, 5), jnp.float32)],
                         axis=1).astype(jnp.bfloat16)

    def step(c, carry):
        bv, bi = carry
        base = c * cblk
        pch = pt_ref[:, pl.ds(base, cblk)]
        pn = pch[0] * pch[0] + pch[1] * pch[1] + pch[2] * pch[2]
        cross = lax.dot_general(q8, pb_ref[:, pl.ds(base, cblk)],
                                (((1,), (0,)), ((), ())),
                                preferred_element_type=jnp.float32)
        d = jnp.maximum(qn + pn[None, :] - 2.0 * cross, 0.0)
        ids = base + lax.broadcasted_iota(jnp.int32, (qblk, cblk),
                                          1).astype(jnp.float32)
        d = jnp.where(ids < n_real, d, jnp.inf)
        cv, ci = _extract8(d, ids)
        mv, mi = _extract8(jnp.concatenate([bv, cv], axis=1),
                           jnp.concatenate([bi, ci], axis=1))
        return mv, mi

    bv0 = jnp.full((qblk, _K), jnp.inf, dtype=jnp.float32)
    bi0 = _INIT_ID + lax.broadcasted_iota(jnp.int32, (qblk, _K),
                                          1).astype(jnp.float32)
    bv, bi = lax.fori_loop(0, n_chunks, step, (bv0, bi0))
    v_ref[...] = bv
    i_ref[...] = bi.astype(jnp.int32)


def _run_topk(positions, coords):
    n = positions.shape[0]
    qn = coords.shape[0]
    cblk = 10240 if n >= 10240 else 128
    n_chunks = -(-n // cblk)
    npad = n_chunks * cblk
    pt = jnp.pad(jnp.transpose(positions), ((0, 0), (0, npad - n)))
    pb = jnp.pad(pt, ((0, 5), (0, 0))).astype(jnp.bfloat16)
    qblk = 128 if qn % 128 == 0 else qn
    grid = (qn // qblk,)
    body = functools.partial(_topk_body, n_real=n, cblk=cblk,
                             n_chunks=n_chunks)
    return pl.pallas_call(
        body,
        grid=grid,
        in_specs=[
            pl.BlockSpec((3, npad), lambda g: (0, 0)),
            pl.BlockSpec((8, npad), lambda g: (0, 0)),
            pl.BlockSpec((qblk, 3), lambda g: (g, 0)),
        ],
        out_specs=[
            pl.BlockSpec((qblk, _K), lambda g: (g, 0)),
            pl.BlockSpec((qblk, _K), lambda g: (g, 0)),
        ],
        out_shape=[
            jax.ShapeDtypeStruct((qn, _K), jnp.float32),
            jax.ShapeDtypeStruct((qn, _K), jnp.int32),
        ],
        compiler_params=pltpu.CompilerParams(
            dimension_semantics=("arbitrary",)),
    )(pt, pb, coords)


def kernel(positions, coords, features):
    positions = positions.astype(jnp.float32)
    coords = coords.astype(jnp.float32)
    topv, topi = _run_topk(positions, coords)
    w = jax.nn.softmax(-topv / _TEMP, axis=1)
    out = jnp.sum(w[..., None] * features[topi], axis=1)
    return out.astype(coords.dtype)
